# baseline (speedup 1.0000x reference)
import functools

import jax
import jax.numpy as jnp
from jax import lax
from jax.experimental import pallas as pl
from jax.experimental.pallas import tpu as pltpu
from jax.experimental.pallas import tpu_sc as plsc

N_FIELDS = 26
VOCAB = 100000
EMB_DIM = 32
B = 1024
L = 50

_info = plsc.get_sparse_core_info()
_NC, _NS, _LANES = _info.num_cores, _info.num_subcores, _info.num_lanes
_NW = _NC * _NS
_UNITS = N_FIELDS * L
_K = -(-_UNITS // _NW)
_JB = B // 128
_DT = EMB_DIM // 8


_VPAD = 100096
_NJ = _VPAD // 128
_U1 = N_FIELDS * _NJ
_K1 = -(-_U1 // _NW)


def _make_sc_relayout():
    mesh = plsc.VectorSubcoreMesh(core_axis_name="c", subcore_axis_name="s")

    @functools.partial(
        pl.kernel,
        mesh=mesh,
        compiler_params=pltpu.CompilerParams(
            use_tc_tiling_on_sc=True, needs_layout_passes=False),
        out_type=jax.ShapeDtypeStruct((N_FIELDS, _VPAD // 4, 128),
                                      jnp.float32),
        scratch_types=[
            pltpu.VMEM((EMB_DIM, 128), jnp.float32),
            pltpu.VMEM((EMB_DIM, 128), jnp.float32),
            pltpu.VMEM((EMB_DIM, 128), jnp.float32),
            pltpu.VMEM((EMB_DIM, 128), jnp.float32),
            pltpu.SemaphoreType.DMA,
            pltpu.SemaphoreType.DMA,
        ],
    )
    def k(tt_hbm, out_hbm, buf0, buf1, st0, st1, sem0, sem1):
        wid = lax.axis_index("s") * _NC + lax.axis_index("c")
        iota16 = lax.iota(jnp.int32, _LANES)
        dvecs = [iota16, _LANES + iota16]

        def fire(u, buf, sem):
            f = u // _NJ
            j = u % _NJ
            pltpu.async_copy(tt_hbm.at[f, :, pl.ds(j * 128, 128)], buf, sem)

        def process(u, buf, st, sem):
            f = u // _NJ
            j = u % _NJ
            pltpu.make_async_copy(
                tt_hbm.at[f, :, pl.ds(j * 128, 128)], buf, sem).wait()

            @plsc.parallel_loop(0, 128, unroll=2)
            def _t(c):
                csplat = jnp.full((_LANES,), c, jnp.int32)
                r_ = c // 4
                for kk in range(2):
                    v = plsc.load_gather(buf, [dvecs[kk], csplat])
                    st[r_, pl.ds((c % 4) * EMB_DIM + kk * _LANES,
                                 _LANES)] = v

            pltpu.sync_copy(st, out_hbm.at[f, pl.ds(32 * j, 32)])

        def unit_of(m):
            return wid + _NW * m

        fire(unit_of(0), buf0, sem0)

        def body(m, carry):
            j0 = 2 * m
            j1 = 2 * m + 1
            j2 = 2 * m + 2

            @pl.when((j1 < _K1) & (unit_of(j1) < _U1))
            def _p1():
                fire(unit_of(j1), buf1, sem1)

            @pl.when(unit_of(j0) < _U1)
            def _d0():
                process(unit_of(j0), buf0, st0, sem0)

            @pl.when((j2 < _K1) & (unit_of(j2) < _U1))
            def _p2():
                fire(unit_of(j2), buf0, sem0)

            @pl.when((j1 < _K1) & (unit_of(j1) < _U1))
            def _d1():
                process(unit_of(j1), buf1, st1, sem1)

            return carry

        lax.fori_loop(0, (_K1 + 1) // 2, body, 0)

    return k


def _make_sc_gather():
    mesh = plsc.VectorSubcoreMesh(core_axis_name="c", subcore_axis_name="s")

    @functools.partial(
        pl.kernel,
        mesh=mesh,
        compiler_params=pltpu.CompilerParams(
            use_tc_tiling_on_sc=False, needs_layout_passes=False),
        out_type=jax.ShapeDtypeStruct((L, N_FIELDS, _DT, _JB, 8, 128),
                                      jnp.float32),
        scratch_types=[
            pltpu.VMEM((B,), jnp.int32),
            pltpu.VMEM((B,), jnp.int32),
            pltpu.VMEM((B, EMB_DIM), jnp.float32),
            pltpu.VMEM((B, EMB_DIM), jnp.float32),
            pltpu.VMEM((_DT, _JB, 8, 128), jnp.float32),
            pltpu.SemaphoreType.DMA,
            pltpu.SemaphoreType.DMA,
        ],
    )
    def k(tables_hbm, xc_hbm, out_hbm,
          idx0_v, idx1_v, rows0_v, rows1_v, stage_v, sem0, sem1):
        wid = lax.axis_index("s") * _NC + lax.axis_index("c")
        iota16 = lax.iota(jnp.int32, _LANES)

        def unit_of(j):
            return wid + _NW * j

        def prefetch(u, idx_v, rows_v, sem):
            f = u // L
            l = u % L
            pltpu.sync_copy(xc_hbm.at[f, l], idx_v)
            foff = f * _VPAD
            for s in range(B // _LANES):
                idx_v[pl.ds(s * _LANES, _LANES)] = (
                    idx_v[pl.ds(s * _LANES, _LANES)] + foff)
            for jb in range(_JB):
                pltpu.async_copy(
                    tables_hbm.at[idx_v.at[pl.ds(jb * 128, 128)]],
                    rows_v.at[pl.ds(jb * 128, 128)], sem)

        def process(u, idx_v, rows_v, sem):
            f = u // L
            l = u % L
            for jb in range(_JB):
                pltpu.make_async_copy(
                    tables_hbm.at[idx_v.at[pl.ds(jb * 128, 128)]],
                    rows_v.at[pl.ds(jb * 128, 128)], sem).wait()

            @plsc.parallel_loop(0, B // _LANES, unroll=2)
            def _t(t):
                bvec = t * _LANES + iota16
                jb = t // 8
                cl = t % 8
                for d in range(EMB_DIM):
                    dsplat = jnp.full((_LANES,), d, jnp.int32)
                    v = plsc.load_gather(rows_v, [bvec, dsplat])
                    stage_v[d // 8, jb, d % 8,
                            pl.ds(cl * _LANES, _LANES)] = v

            pltpu.sync_copy(stage_v, out_hbm.at[l, f])

        prefetch(unit_of(0), idx0_v, rows0_v, sem0)

        def body(m, carry):
            j0 = 2 * m
            j1 = 2 * m + 1
            j2 = 2 * m + 2

            @pl.when((j1 < _K) & (unit_of(j1) < _UNITS))
            def _p1():
                prefetch(unit_of(j1), idx1_v, rows1_v, sem1)

            @pl.when(unit_of(j0) < _UNITS)
            def _d0():
                process(unit_of(j0), idx0_v, rows0_v, sem0)

            @pl.when((j2 < _K) & (unit_of(j2) < _UNITS))
            def _p2():
                prefetch(unit_of(j2), idx0_v, rows0_v, sem0)

            @pl.when((j1 < _K) & (unit_of(j1) < _UNITS))
            def _d1():
                process(unit_of(j1), idx1_v, rows1_v, sem1)

            return carry

        lax.fori_loop(0, (_K + 1) // 2, body, 0)

    return k


_sc_gather = _make_sc_gather()
_sc_relayout = _make_sc_relayout()


def kernel(x_cat, tables):
    tt = jnp.transpose(tables, (0, 2, 1))
    t128 = _sc_relayout(tt)
    flat_tables = t128.reshape(N_FIELDS * _VPAD, EMB_DIM)
    xc = jnp.transpose(x_cat, (2, 1, 0)).astype(jnp.int32)
    o6 = _sc_gather(flat_tables, xc)
    o = o6.transpose(0, 1, 2, 4, 3, 5).reshape(L, N_FIELDS, EMB_DIM, B)
    return o.transpose(3, 0, 1, 2)


# speedup vs baseline: 1.0012x; 1.0012x over previous
import functools

import jax
import jax.numpy as jnp
from jax import lax
from jax.experimental import pallas as pl
from jax.experimental.pallas import tpu as pltpu
from jax.experimental.pallas import tpu_sc as plsc

N_FIELDS = 26
VOCAB = 100000
EMB_DIM = 32
B = 1024
L = 50

_info = plsc.get_sparse_core_info()
_NC, _NS, _LANES = _info.num_cores, _info.num_subcores, _info.num_lanes
_NW = _NC * _NS
_UNITS = N_FIELDS * L
_K = -(-_UNITS // _NW)
_JB = B // 128
_DT = EMB_DIM // 8


_VPAD = 100096
_NJ = _VPAD // 256
_U1 = N_FIELDS * _NJ
_K1 = -(-_U1 // _NW)


def _make_sc_relayout():
    mesh = plsc.VectorSubcoreMesh(core_axis_name="c", subcore_axis_name="s")

    @functools.partial(
        pl.kernel,
        mesh=mesh,
        compiler_params=pltpu.CompilerParams(
            use_tc_tiling_on_sc=True, needs_layout_passes=False),
        out_type=jax.ShapeDtypeStruct((N_FIELDS, _VPAD // 4, 128),
                                      jnp.float32),
        scratch_types=[
            pltpu.VMEM((EMB_DIM, 256), jnp.float32),
            pltpu.VMEM((EMB_DIM, 256), jnp.float32),
            pltpu.VMEM((64, 128), jnp.float32),
            pltpu.VMEM((64, 128), jnp.float32),
            pltpu.SemaphoreType.DMA,
            pltpu.SemaphoreType.DMA,
        ],
    )
    def k(tt_hbm, out_hbm, buf0, buf1, st0, st1, sem0, sem1):
        wid = lax.axis_index("s") * _NC + lax.axis_index("c")
        iota16 = lax.iota(jnp.int32, _LANES)
        dvecs = [iota16, _LANES + iota16]

        def fire(u, buf, sem):
            f = u // _NJ
            j = u % _NJ
            pltpu.async_copy(tt_hbm.at[f, :, pl.ds(j * 256, 256)], buf, sem)

        def process(u, buf, st, sem):
            f = u // _NJ
            j = u % _NJ
            pltpu.make_async_copy(
                tt_hbm.at[f, :, pl.ds(j * 256, 256)], buf, sem).wait()

            @plsc.parallel_loop(0, 256, unroll=4)
            def _t(c):
                csplat = jnp.full((_LANES,), c, jnp.int32)
                r_ = c // 4
                for kk in range(2):
                    v = plsc.load_gather(buf, [dvecs[kk], csplat])
                    st[r_, pl.ds((c % 4) * EMB_DIM + kk * _LANES,
                                 _LANES)] = v

            pltpu.sync_copy(st, out_hbm.at[f, pl.ds(64 * j, 64)])

        def unit_of(m):
            return wid + _NW * m

        fire(unit_of(0), buf0, sem0)

        def body(m, carry):
            j0 = 2 * m
            j1 = 2 * m + 1
            j2 = 2 * m + 2

            @pl.when((j1 < _K1) & (unit_of(j1) < _U1))
            def _p1():
                fire(unit_of(j1), buf1, sem1)

            @pl.when(unit_of(j0) < _U1)
            def _d0():
                process(unit_of(j0), buf0, st0, sem0)

            @pl.when((j2 < _K1) & (unit_of(j2) < _U1))
            def _p2():
                fire(unit_of(j2), buf0, sem0)

            @pl.when((j1 < _K1) & (unit_of(j1) < _U1))
            def _d1():
                process(unit_of(j1), buf1, st1, sem1)

            return carry

        lax.fori_loop(0, (_K1 + 1) // 2, body, 0)

    return k


def _make_sc_gather():
    mesh = plsc.VectorSubcoreMesh(core_axis_name="c", subcore_axis_name="s")

    @functools.partial(
        pl.kernel,
        mesh=mesh,
        compiler_params=pltpu.CompilerParams(
            use_tc_tiling_on_sc=False, needs_layout_passes=False),
        out_type=jax.ShapeDtypeStruct((L, N_FIELDS, _DT, _JB, 8, 128),
                                      jnp.float32),
        scratch_types=[
            pltpu.VMEM((B,), jnp.int32),
            pltpu.VMEM((B,), jnp.int32),
            pltpu.VMEM((B, EMB_DIM), jnp.float32),
            pltpu.VMEM((B, EMB_DIM), jnp.float32),
            pltpu.VMEM((_DT, _JB, 8, 128), jnp.float32),
            pltpu.SemaphoreType.DMA,
            pltpu.SemaphoreType.DMA,
        ],
    )
    def k(tables_hbm, xc_hbm, out_hbm,
          idx0_v, idx1_v, rows0_v, rows1_v, stage_v, sem0, sem1):
        wid = lax.axis_index("s") * _NC + lax.axis_index("c")
        iota16 = lax.iota(jnp.int32, _LANES)

        def unit_of(j):
            return wid + _NW * j

        def prefetch(u, idx_v, rows_v, sem):
            f = u // L
            l = u % L
            pltpu.sync_copy(xc_hbm.at[f, l], idx_v)
            foff = f * _VPAD
            for s in range(B // _LANES):
                idx_v[pl.ds(s * _LANES, _LANES)] = (
                    idx_v[pl.ds(s * _LANES, _LANES)] + foff)
            for jb in range(_JB):
                pltpu.async_copy(
                    tables_hbm.at[idx_v.at[pl.ds(jb * 128, 128)]],
                    rows_v.at[pl.ds(jb * 128, 128)], sem)

        def process(u, idx_v, rows_v, sem):
            f = u // L
            l = u % L
            for jb in range(_JB):
                pltpu.make_async_copy(
                    tables_hbm.at[idx_v.at[pl.ds(jb * 128, 128)]],
                    rows_v.at[pl.ds(jb * 128, 128)], sem).wait()

            @plsc.parallel_loop(0, B // _LANES, unroll=2)
            def _t(t):
                bvec = t * _LANES + iota16
                jb = t // 8
                cl = t % 8
                for d in range(EMB_DIM):
                    dsplat = jnp.full((_LANES,), d, jnp.int32)
                    v = plsc.load_gather(rows_v, [bvec, dsplat])
                    stage_v[d // 8, jb, d % 8,
                            pl.ds(cl * _LANES, _LANES)] = v

            pltpu.sync_copy(stage_v, out_hbm.at[l, f])

        prefetch(unit_of(0), idx0_v, rows0_v, sem0)

        def body(m, carry):
            j0 = 2 * m
            j1 = 2 * m + 1
            j2 = 2 * m + 2

            @pl.when((j1 < _K) & (unit_of(j1) < _UNITS))
            def _p1():
                prefetch(unit_of(j1), idx1_v, rows1_v, sem1)

            @pl.when(unit_of(j0) < _UNITS)
            def _d0():
                process(unit_of(j0), idx0_v, rows0_v, sem0)

            @pl.when((j2 < _K) & (unit_of(j2) < _UNITS))
            def _p2():
                prefetch(unit_of(j2), idx0_v, rows0_v, sem0)

            @pl.when((j1 < _K) & (unit_of(j1) < _UNITS))
            def _d1():
                process(unit_of(j1), idx1_v, rows1_v, sem1)

            return carry

        lax.fori_loop(0, (_K + 1) // 2, body, 0)

    return k


_sc_gather = _make_sc_gather()
_sc_relayout = _make_sc_relayout()


def kernel(x_cat, tables):
    tt = jnp.transpose(tables, (0, 2, 1))
    t128 = _sc_relayout(tt)
    flat_tables = t128.reshape(N_FIELDS * _VPAD, EMB_DIM)
    xc = jnp.transpose(x_cat, (2, 1, 0)).astype(jnp.int32)
    o6 = _sc_gather(flat_tables, xc)
    o = o6.transpose(0, 1, 2, 4, 3, 5).reshape(L, N_FIELDS, EMB_DIM, B)
    return o.transpose(3, 0, 1, 2)


# speedup vs baseline: 1.3480x; 1.3464x over previous
import functools

import jax
import jax.numpy as jnp
from jax import lax
from jax.experimental import pallas as pl
from jax.experimental.pallas import tpu as pltpu
from jax.experimental.pallas import tpu_sc as plsc

N_FIELDS = 26
VOCAB = 100000
EMB_DIM = 32
B = 1024
L = 50

_info = plsc.get_sparse_core_info()
_NC, _NS, _LANES = _info.num_cores, _info.num_subcores, _info.num_lanes
_NW = _NC * _NS
_UNITS = N_FIELDS * L
_K = -(-_UNITS // _NW)
_JB = B // 128
_DT = EMB_DIM // 8


_VPAD = 100096
_NJ = _VPAD // 256
_U1 = N_FIELDS * _NJ
_K1 = -(-_U1 // _NW)


def _make_sc_relayout():
    mesh = plsc.VectorSubcoreMesh(core_axis_name="c", subcore_axis_name="s")

    @functools.partial(
        pl.kernel,
        mesh=mesh,
        compiler_params=pltpu.CompilerParams(
            use_tc_tiling_on_sc=True, needs_layout_passes=False),
        out_type=jax.ShapeDtypeStruct((N_FIELDS, _VPAD // 4, 128),
                                      jnp.float32),
        scratch_types=[
            pltpu.VMEM((EMB_DIM, 257), jnp.float32),
            pltpu.VMEM((EMB_DIM, 257), jnp.float32),
            pltpu.VMEM((64, 128), jnp.float32),
            pltpu.VMEM((64, 128), jnp.float32),
            pltpu.SemaphoreType.DMA,
            pltpu.SemaphoreType.DMA,
        ],
    )
    def k(tt_hbm, out_hbm, buf0, buf1, st0, st1, sem0, sem1):
        wid = lax.axis_index("s") * _NC + lax.axis_index("c")
        iota16 = lax.iota(jnp.int32, _LANES)
        dvecs = [iota16, _LANES + iota16]

        def fire(u, buf, sem):
            f = u // _NJ
            j = u % _NJ
            pltpu.async_copy(tt_hbm.at[f, :, pl.ds(j * 256, 256)],
                             buf.at[:, pl.ds(0, 256)], sem)

        def process(u, buf, st, sem):
            f = u // _NJ
            j = u % _NJ
            pltpu.make_async_copy(
                tt_hbm.at[f, :, pl.ds(j * 256, 256)],
                buf.at[:, pl.ds(0, 256)], sem).wait()

            @plsc.parallel_loop(0, 256, unroll=4)
            def _t(c):
                csplat = jnp.full((_LANES,), c, jnp.int32)
                r_ = c // 4
                for kk in range(2):
                    v = plsc.load_gather(buf, [dvecs[kk], csplat])
                    st[r_, pl.ds((c % 4) * EMB_DIM + kk * _LANES,
                                 _LANES)] = v

            pltpu.sync_copy(st, out_hbm.at[f, pl.ds(64 * j, 64)])

        def unit_of(m):
            return wid + _NW * m

        fire(unit_of(0), buf0, sem0)

        def body(m, carry):
            j0 = 2 * m
            j1 = 2 * m + 1
            j2 = 2 * m + 2

            @pl.when((j1 < _K1) & (unit_of(j1) < _U1))
            def _p1():
                fire(unit_of(j1), buf1, sem1)

            @pl.when(unit_of(j0) < _U1)
            def _d0():
                process(unit_of(j0), buf0, st0, sem0)

            @pl.when((j2 < _K1) & (unit_of(j2) < _U1))
            def _p2():
                fire(unit_of(j2), buf0, sem0)

            @pl.when((j1 < _K1) & (unit_of(j1) < _U1))
            def _d1():
                process(unit_of(j1), buf1, st1, sem1)

            return carry

        lax.fori_loop(0, (_K1 + 1) // 2, body, 0)

    return k


def _make_sc_gather():
    mesh = plsc.VectorSubcoreMesh(core_axis_name="c", subcore_axis_name="s")

    @functools.partial(
        pl.kernel,
        mesh=mesh,
        compiler_params=pltpu.CompilerParams(
            use_tc_tiling_on_sc=False, needs_layout_passes=False),
        out_type=jax.ShapeDtypeStruct((L, N_FIELDS, _DT * _JB * 8 * 128),
                                      jnp.float32),
        scratch_types=[
            pltpu.VMEM((B,), jnp.int32),
            pltpu.VMEM((B,), jnp.int32),
            pltpu.VMEM((B, EMB_DIM), jnp.float32),
            pltpu.VMEM((B, EMB_DIM), jnp.float32),
            pltpu.VMEM((_DT * _JB * 8 * 128,), jnp.float32),
            pltpu.SemaphoreType.DMA,
            pltpu.SemaphoreType.DMA,
        ],
    )
    def k(tables_hbm, xc_hbm, out_hbm,
          idx0_v, idx1_v, rows0_v, rows1_v, stage_v, sem0, sem1):
        wid = lax.axis_index("s") * _NC + lax.axis_index("c")
        iota16 = lax.iota(jnp.int32, _LANES)

        def unit_of(j):
            return wid + _NW * j

        def prefetch(u, idx_v, rows_v, sem):
            f = u // L
            l = u % L
            pltpu.sync_copy(xc_hbm.at[f, l], idx_v)
            foff = f * _VPAD
            for s in range(B // _LANES):
                idx_v[pl.ds(s * _LANES, _LANES)] = (
                    idx_v[pl.ds(s * _LANES, _LANES)] + foff)
            for jb in range(_JB):
                pltpu.async_copy(
                    tables_hbm.at[idx_v.at[pl.ds(jb * 128, 128)]],
                    rows_v.at[pl.ds(jb * 128, 128)], sem)

        def drain(idx_v, rows_v, sem):
            for jb in range(_JB):
                pltpu.make_async_copy(
                    tables_hbm.at[idx_v.at[pl.ds(jb * 128, 128)]],
                    rows_v.at[pl.ds(jb * 128, 128)], sem).wait()

        def transpose_store(u, rows_v):
            f = u // L
            l = u % L

            @plsc.parallel_loop(0, B // _LANES, unroll=2)
            def _t(t):
                bvec = t * _LANES + iota16
                qt = (t // 8) * 1024 + (t % 8) * _LANES + iota16
                for s in range(_LANES):
                    rot = (iota16 + s) % _LANES
                    p = qt + (rot // 8) * 8192 + (rot % 8) * 128
                    for kk in range(2):
                        dvec = rot + kk * _LANES
                        v = plsc.load_gather(rows_v, [bvec, dvec])
                        plsc.store_scatter(stage_v, [p + kk * 16384], v)

            pltpu.sync_copy(stage_v, out_hbm.at[l, f])

        prefetch(unit_of(0), idx0_v, rows0_v, sem0)

        def body(m, carry):
            j0 = 2 * m
            j1 = 2 * m + 1
            j2 = 2 * m + 2

            @pl.when((j1 < _K) & (unit_of(j1) < _UNITS))
            def _p1():
                prefetch(unit_of(j1), idx1_v, rows1_v, sem1)

            @pl.when(unit_of(j0) < _UNITS)
            def _d0():
                drain(idx0_v, rows0_v, sem0)
                transpose_store(unit_of(j0), rows0_v)

            @pl.when((j2 < _K) & (unit_of(j2) < _UNITS))
            def _p2():
                prefetch(unit_of(j2), idx0_v, rows0_v, sem0)

            @pl.when((j1 < _K) & (unit_of(j1) < _UNITS))
            def _d1():
                drain(idx1_v, rows1_v, sem1)
                transpose_store(unit_of(j1), rows1_v)

            return carry

        lax.fori_loop(0, (_K + 1) // 2, body, 0)

    return k


_sc_gather = _make_sc_gather()
_sc_relayout = _make_sc_relayout()


def kernel(x_cat, tables):
    tt = jnp.transpose(tables, (0, 2, 1))
    t128 = _sc_relayout(tt)
    flat_tables = t128.reshape(N_FIELDS * _VPAD, EMB_DIM)
    xc = jnp.transpose(x_cat, (2, 1, 0)).astype(jnp.int32)
    o6 = _sc_gather(flat_tables, xc).reshape(L, N_FIELDS, _DT, _JB, 8, 128)
    o = o6.transpose(0, 1, 2, 4, 3, 5).reshape(L, N_FIELDS, EMB_DIM, B)
    return o.transpose(3, 0, 1, 2)


# speedup vs baseline: 1.4673x; 1.0885x over previous
import functools

import jax
import jax.numpy as jnp
from jax import lax
from jax.experimental import pallas as pl
from jax.experimental.pallas import tpu as pltpu
from jax.experimental.pallas import tpu_sc as plsc

N_FIELDS = 26
VOCAB = 100000
EMB_DIM = 32
B = 1024
L = 50

_info = plsc.get_sparse_core_info()
_NC, _NS, _LANES = _info.num_cores, _info.num_subcores, _info.num_lanes
_NW = _NC * _NS
_UNITS = N_FIELDS * L
_K = -(-_UNITS // _NW)
_JB = B // 128
_DT = EMB_DIM // 8


_VPAD = 100096
_NJ = _VPAD // 256
_U1 = N_FIELDS * _NJ
_K1 = -(-_U1 // _NW)


def _make_sc_relayout():
    mesh = plsc.VectorSubcoreMesh(core_axis_name="c", subcore_axis_name="s")

    @functools.partial(
        pl.kernel,
        mesh=mesh,
        compiler_params=pltpu.CompilerParams(
            use_tc_tiling_on_sc=True, needs_layout_passes=False),
        out_type=jax.ShapeDtypeStruct((N_FIELDS, _VPAD // 4, 128),
                                      jnp.float32),
        scratch_types=[
            pltpu.VMEM((EMB_DIM, 257), jnp.float32),
            pltpu.VMEM((EMB_DIM, 257), jnp.float32),
            pltpu.VMEM((64, 128), jnp.float32),
            pltpu.VMEM((64, 128), jnp.float32),
            pltpu.SemaphoreType.DMA,
            pltpu.SemaphoreType.DMA,
            pltpu.SemaphoreType.DMA,
            pltpu.SemaphoreType.DMA,
        ],
    )
    def k(tt_hbm, out_hbm, buf0, buf1, st0, st1, sem0, sem1, osem0, osem1):
        wid = lax.axis_index("s") * _NC + lax.axis_index("c")
        iota16 = lax.iota(jnp.int32, _LANES)
        dvecs = [iota16, _LANES + iota16]

        def fire(u, buf, sem):
            f = u // _NJ
            j = u % _NJ
            pltpu.async_copy(tt_hbm.at[f, :, pl.ds(j * 256, 256)],
                             buf.at[:, pl.ds(0, 256)], sem)

        def out_ref_of(u):
            f = u // _NJ
            j = u % _NJ
            return out_hbm.at[f, pl.ds(64 * j, 64)]

        def process(u, buf, st, sem, osem, first):
            f = u // _NJ
            j = u % _NJ
            pltpu.make_async_copy(
                tt_hbm.at[f, :, pl.ds(j * 256, 256)],
                buf.at[:, pl.ds(0, 256)], sem).wait()

            @pl.when(jnp.logical_not(first))
            def _wo():
                pltpu.make_async_copy(st, out_ref_of(u), osem).wait()

            @plsc.parallel_loop(0, 256, unroll=4)
            def _t(c):
                csplat = jnp.full((_LANES,), c, jnp.int32)
                r_ = c // 4
                for kk in range(2):
                    v = plsc.load_gather(buf, [dvecs[kk], csplat])
                    st[r_, pl.ds((c % 4) * EMB_DIM + kk * _LANES,
                                 _LANES)] = v

            pltpu.async_copy(st, out_ref_of(u), osem)

        def unit_of(m):
            return wid + _NW * m

        fire(unit_of(0), buf0, sem0)

        def body(m, carry):
            j0 = 2 * m
            j1 = 2 * m + 1
            j2 = 2 * m + 2

            @pl.when((j1 < _K1) & (unit_of(j1) < _U1))
            def _p1():
                fire(unit_of(j1), buf1, sem1)

            @pl.when(unit_of(j0) < _U1)
            def _d0():
                process(unit_of(j0), buf0, st0, sem0, osem0, m == 0)

            @pl.when((j2 < _K1) & (unit_of(j2) < _U1))
            def _p2():
                fire(unit_of(j2), buf0, sem0)

            @pl.when((j1 < _K1) & (unit_of(j1) < _U1))
            def _d1():
                process(unit_of(j1), buf1, st1, sem1, osem1, m == 0)

            return carry

        lax.fori_loop(0, (_K1 + 1) // 2, body, 0)

        pltpu.make_async_copy(st0, out_ref_of(0), osem0).wait()
        pltpu.make_async_copy(st1, out_ref_of(0), osem1).wait()

    return k


def _make_sc_gather():
    mesh = plsc.VectorSubcoreMesh(core_axis_name="c", subcore_axis_name="s")

    @functools.partial(
        pl.kernel,
        mesh=mesh,
        compiler_params=pltpu.CompilerParams(
            use_tc_tiling_on_sc=False, needs_layout_passes=False),
        out_type=jax.ShapeDtypeStruct((L, N_FIELDS, _DT * _JB * 8 * 128),
                                      jnp.float32),
        scratch_types=[
            pltpu.VMEM((B,), jnp.int32),
            pltpu.VMEM((B,), jnp.int32),
            pltpu.VMEM((B, EMB_DIM), jnp.float32),
            pltpu.VMEM((B, EMB_DIM), jnp.float32),
            pltpu.VMEM((_DT * _JB * 8 * 128,), jnp.float32),
            pltpu.SemaphoreType.DMA,
            pltpu.SemaphoreType.DMA,
        ],
    )
    def k(tables_hbm, xc_hbm, out_hbm,
          idx0_v, idx1_v, rows0_v, rows1_v, stage_v, sem0, sem1):
        wid = lax.axis_index("s") * _NC + lax.axis_index("c")
        iota16 = lax.iota(jnp.int32, _LANES)

        def unit_of(j):
            return wid + _NW * j

        def prefetch(u, idx_v, rows_v, sem):
            f = u // L
            l = u % L
            pltpu.sync_copy(xc_hbm.at[f, l], idx_v)
            foff = f * _VPAD
            for s in range(B // _LANES):
                idx_v[pl.ds(s * _LANES, _LANES)] = (
                    idx_v[pl.ds(s * _LANES, _LANES)] + foff)
            for jb in range(_JB):
                pltpu.async_copy(
                    tables_hbm.at[idx_v.at[pl.ds(jb * 128, 128)]],
                    rows_v.at[pl.ds(jb * 128, 128)], sem)

        def drain(idx_v, rows_v, sem):
            for jb in range(_JB):
                pltpu.make_async_copy(
                    tables_hbm.at[idx_v.at[pl.ds(jb * 128, 128)]],
                    rows_v.at[pl.ds(jb * 128, 128)], sem).wait()

        def transpose_store(u, rows_v):
            f = u // L
            l = u % L

            @plsc.parallel_loop(0, B // _LANES, unroll=2)
            def _t(t):
                bvec = t * _LANES + iota16
                qt = (t // 8) * 1024 + (t % 8) * _LANES + iota16
                for s in range(_LANES):
                    rot = (iota16 + s) % _LANES
                    p = qt + (rot // 8) * 8192 + (rot % 8) * 128
                    for kk in range(2):
                        dvec = rot + kk * _LANES
                        v = plsc.load_gather(rows_v, [bvec, dvec])
                        plsc.store_scatter(stage_v, [p + kk * 16384], v)

            pltpu.sync_copy(stage_v, out_hbm.at[l, f])

        prefetch(unit_of(0), idx0_v, rows0_v, sem0)

        def body(m, carry):
            j0 = 2 * m
            j1 = 2 * m + 1
            j2 = 2 * m + 2

            @pl.when((j1 < _K) & (unit_of(j1) < _UNITS))
            def _p1():
                prefetch(unit_of(j1), idx1_v, rows1_v, sem1)

            @pl.when(unit_of(j0) < _UNITS)
            def _d0():
                drain(idx0_v, rows0_v, sem0)
                transpose_store(unit_of(j0), rows0_v)

            @pl.when((j2 < _K) & (unit_of(j2) < _UNITS))
            def _p2():
                prefetch(unit_of(j2), idx0_v, rows0_v, sem0)

            @pl.when((j1 < _K) & (unit_of(j1) < _UNITS))
            def _d1():
                drain(idx1_v, rows1_v, sem1)
                transpose_store(unit_of(j1), rows1_v)

            return carry

        lax.fori_loop(0, (_K + 1) // 2, body, 0)

    return k


_sc_gather = _make_sc_gather()
_sc_relayout = _make_sc_relayout()


def kernel(x_cat, tables):
    tt = jnp.transpose(tables, (0, 2, 1))
    t128 = _sc_relayout(tt)
    flat_tables = t128.reshape(N_FIELDS * _VPAD, EMB_DIM)
    xc = jnp.transpose(x_cat, (2, 1, 0)).astype(jnp.int32)
    o6 = _sc_gather(flat_tables, xc).reshape(L, N_FIELDS, _DT, _JB, 8, 128)
    o = o6.transpose(0, 1, 2, 4, 3, 5).reshape(L, N_FIELDS, EMB_DIM, B)
    return o.transpose(3, 0, 1, 2)


# speedup vs baseline: 1.4710x; 1.0025x over previous
import functools

import jax
import jax.numpy as jnp
from jax import lax
from jax.experimental import pallas as pl
from jax.experimental.pallas import tpu as pltpu
from jax.experimental.pallas import tpu_sc as plsc

N_FIELDS = 26
VOCAB = 100000
EMB_DIM = 32
B = 1024
L = 50

_info = plsc.get_sparse_core_info()
_NC, _NS, _LANES = _info.num_cores, _info.num_subcores, _info.num_lanes
_NW = _NC * _NS
_UNITS = N_FIELDS * L
_K = -(-_UNITS // _NW)
_JB = B // 128
_DT = EMB_DIM // 8


_VPAD = 100096
_NJ = _VPAD // 256
_U1 = N_FIELDS * _NJ
_K1 = -(-_U1 // _NW)


def _make_sc_relayout():
    mesh = plsc.VectorSubcoreMesh(core_axis_name="c", subcore_axis_name="s")

    @functools.partial(
        pl.kernel,
        mesh=mesh,
        compiler_params=pltpu.CompilerParams(
            use_tc_tiling_on_sc=True, needs_layout_passes=False),
        out_type=jax.ShapeDtypeStruct((N_FIELDS, _VPAD // 4, 128),
                                      jnp.float32),
        scratch_types=[
            pltpu.VMEM((EMB_DIM, 257), jnp.float32),
            pltpu.VMEM((EMB_DIM, 257), jnp.float32),
            pltpu.VMEM((64, 128), jnp.float32),
            pltpu.VMEM((64, 128), jnp.float32),
            pltpu.SemaphoreType.DMA,
            pltpu.SemaphoreType.DMA,
            pltpu.SemaphoreType.DMA,
            pltpu.SemaphoreType.DMA,
        ],
    )
    def k(tt_hbm, out_hbm, buf0, buf1, st0, st1, sem0, sem1, osem0, osem1):
        wid = lax.axis_index("s") * _NC + lax.axis_index("c")
        iota16 = lax.iota(jnp.int32, _LANES)
        dvecs = [iota16, _LANES + iota16]

        def fire(u, buf, sem):
            f = u // _NJ
            j = u % _NJ
            pltpu.async_copy(tt_hbm.at[f, :, pl.ds(j * 256, 256)],
                             buf.at[:, pl.ds(0, 256)], sem)

        def out_ref_of(u):
            f = u // _NJ
            j = u % _NJ
            return out_hbm.at[f, pl.ds(64 * j, 64)]

        def process(u, buf, st, sem, osem, first):
            f = u // _NJ
            j = u % _NJ
            pltpu.make_async_copy(
                tt_hbm.at[f, :, pl.ds(j * 256, 256)],
                buf.at[:, pl.ds(0, 256)], sem).wait()

            @pl.when(jnp.logical_not(first))
            def _wo():
                pltpu.make_async_copy(st, out_ref_of(u), osem).wait()

            @plsc.parallel_loop(0, 256, unroll=16)
            def _t(c):
                csplat = jnp.full((_LANES,), c, jnp.int32)
                r_ = c // 4
                for kk in range(2):
                    v = plsc.load_gather(buf, [dvecs[kk], csplat])
                    st[r_, pl.ds((c % 4) * EMB_DIM + kk * _LANES,
                                 _LANES)] = v

            pltpu.async_copy(st, out_ref_of(u), osem)

        def unit_of(m):
            return wid + _NW * m

        fire(unit_of(0), buf0, sem0)

        def body(m, carry):
            j0 = 2 * m
            j1 = 2 * m + 1
            j2 = 2 * m + 2

            @pl.when((j1 < _K1) & (unit_of(j1) < _U1))
            def _p1():
                fire(unit_of(j1), buf1, sem1)

            @pl.when(unit_of(j0) < _U1)
            def _d0():
                process(unit_of(j0), buf0, st0, sem0, osem0, m == 0)

            @pl.when((j2 < _K1) & (unit_of(j2) < _U1))
            def _p2():
                fire(unit_of(j2), buf0, sem0)

            @pl.when((j1 < _K1) & (unit_of(j1) < _U1))
            def _d1():
                process(unit_of(j1), buf1, st1, sem1, osem1, m == 0)

            return carry

        lax.fori_loop(0, (_K1 + 1) // 2, body, 0)

        pltpu.make_async_copy(st0, out_ref_of(0), osem0).wait()
        pltpu.make_async_copy(st1, out_ref_of(0), osem1).wait()

    return k


def _make_sc_gather():
    mesh = plsc.VectorSubcoreMesh(core_axis_name="c", subcore_axis_name="s")

    @functools.partial(
        pl.kernel,
        mesh=mesh,
        compiler_params=pltpu.CompilerParams(
            use_tc_tiling_on_sc=False, needs_layout_passes=False),
        out_type=jax.ShapeDtypeStruct((L, N_FIELDS, _DT * _JB * 8 * 128),
                                      jnp.float32),
        scratch_types=[
            pltpu.VMEM((B,), jnp.int32),
            pltpu.VMEM((B,), jnp.int32),
            pltpu.VMEM((B, EMB_DIM), jnp.float32),
            pltpu.VMEM((B, EMB_DIM), jnp.float32),
            pltpu.VMEM((_DT * _JB * 8 * 128,), jnp.float32),
            pltpu.SemaphoreType.DMA,
            pltpu.SemaphoreType.DMA,
        ],
    )
    def k(tables_hbm, xc_hbm, out_hbm,
          idx0_v, idx1_v, rows0_v, rows1_v, stage_v, sem0, sem1):
        wid = lax.axis_index("s") * _NC + lax.axis_index("c")
        iota16 = lax.iota(jnp.int32, _LANES)

        def unit_of(j):
            return wid + _NW * j

        def prefetch(u, idx_v, rows_v, sem):
            f = u // L
            l = u % L
            pltpu.sync_copy(xc_hbm.at[f, l], idx_v)
            foff = f * _VPAD
            for s in range(B // _LANES):
                idx_v[pl.ds(s * _LANES, _LANES)] = (
                    idx_v[pl.ds(s * _LANES, _LANES)] + foff)
            for jb in range(_JB):
                pltpu.async_copy(
                    tables_hbm.at[idx_v.at[pl.ds(jb * 128, 128)]],
                    rows_v.at[pl.ds(jb * 128, 128)], sem)

        def drain(idx_v, rows_v, sem):
            for jb in range(_JB):
                pltpu.make_async_copy(
                    tables_hbm.at[idx_v.at[pl.ds(jb * 128, 128)]],
                    rows_v.at[pl.ds(jb * 128, 128)], sem).wait()

        def transpose_store(u, rows_v):
            f = u // L
            l = u % L

            @plsc.parallel_loop(0, B // _LANES, unroll=2)
            def _t(t):
                bvec = t * _LANES + iota16
                qt = (t // 8) * 1024 + (t % 8) * _LANES + iota16
                for s in range(_LANES):
                    rot = (iota16 + s) % _LANES
                    p = qt + (rot // 8) * 8192 + (rot % 8) * 128
                    for kk in range(2):
                        dvec = rot + kk * _LANES
                        v = plsc.load_gather(rows_v, [bvec, dvec])
                        plsc.store_scatter(stage_v, [p + kk * 16384], v)

            pltpu.sync_copy(stage_v, out_hbm.at[l, f])

        prefetch(unit_of(0), idx0_v, rows0_v, sem0)

        def body(m, carry):
            j0 = 2 * m
            j1 = 2 * m + 1
            j2 = 2 * m + 2

            @pl.when((j1 < _K) & (unit_of(j1) < _UNITS))
            def _p1():
                prefetch(unit_of(j1), idx1_v, rows1_v, sem1)

            @pl.when(unit_of(j0) < _UNITS)
            def _d0():
                drain(idx0_v, rows0_v, sem0)
                transpose_store(unit_of(j0), rows0_v)

            @pl.when((j2 < _K) & (unit_of(j2) < _UNITS))
            def _p2():
                prefetch(unit_of(j2), idx0_v, rows0_v, sem0)

            @pl.when((j1 < _K) & (unit_of(j1) < _UNITS))
            def _d1():
                drain(idx1_v, rows1_v, sem1)
                transpose_store(unit_of(j1), rows1_v)

            return carry

        lax.fori_loop(0, (_K + 1) // 2, body, 0)

    return k


_sc_gather = _make_sc_gather()
_sc_relayout = _make_sc_relayout()


def kernel(x_cat, tables):
    tt = jnp.transpose(tables, (0, 2, 1))
    t128 = _sc_relayout(tt)
    flat_tables = t128.reshape(N_FIELDS * _VPAD, EMB_DIM)
    xc = jnp.transpose(x_cat, (2, 1, 0)).astype(jnp.int32)
    o6 = _sc_gather(flat_tables, xc).reshape(L, N_FIELDS, _DT, _JB, 8, 128)
    o = o6.transpose(0, 1, 2, 4, 3, 5).reshape(L, N_FIELDS, EMB_DIM, B)
    return o.transpose(3, 0, 1, 2)
